# baseline (speedup 1.0000x reference)
import jax
import jax.numpy as jnp
from jax.experimental import pallas as pl
from jax.experimental.pallas import tpu as pltpu

_E = 8
_BLK = 1024


def _body(h_ref, p_ref, m_ref, out_ref, tpe_ref):
    m = m_ref[...]
    scale = jnp.sum(p_ref[...] * m, axis=1, keepdims=True)
    out_ref[...] = h_ref[...] * scale
    tpe_ref[...] = jnp.sum(m, axis=0, keepdims=True)[None]


def _reduce_body(part_ref, tpe_ref):
    tpe_ref[...] = jnp.sum(part_ref[...], axis=0)


def kernel(hidden_states, probs, routing_map):
    hidden_shape = hidden_states.shape
    H = hidden_shape[-1]
    T = hidden_states.size // H
    nb = T // _BLK
    h = hidden_states.reshape(T, H)
    mask_f = routing_map.astype(jnp.float32)

    out, tpe_parts = pl.pallas_call(
        _body,
        grid=(nb,),
        in_specs=[
            pl.BlockSpec((_BLK, H), lambda i: (i, 0)),
            pl.BlockSpec((_BLK, _E), lambda i: (i, 0)),
            pl.BlockSpec((_BLK, _E), lambda i: (i, 0)),
        ],
        out_specs=[
            pl.BlockSpec((_BLK, H), lambda i: (i, 0)),
            pl.BlockSpec((1, 1, _E), lambda i: (i, 0, 0)),
        ],
        out_shape=[
            jax.ShapeDtypeStruct((T, H), jnp.float32),
            jax.ShapeDtypeStruct((nb, 1, _E), jnp.float32),
        ],
        compiler_params=pltpu.CompilerParams(
            dimension_semantics=("parallel",),
        ),
    )(h, probs, mask_f)

    tpe = pl.pallas_call(
        _reduce_body,
        out_shape=jax.ShapeDtypeStruct((1, _E), jnp.float32),
    )(tpe_parts)

    tokens_per_expert = tpe.reshape(_E).astype(jnp.int64)
    return out.reshape(hidden_shape), tokens_per_expert
